# Initial kernel scaffold; baseline (speedup 1.0000x reference)
import jax
import jax.numpy as jnp
from jax.experimental import pallas as pl


def kernel(x, edge_index, edge_type, W_rel1, W_root1, b1, ln1_g, ln1_b, W_rel2, W_root2, b2, ln2_g, ln2_b, cW1, cb1, cln1_g, cln1_b, cW2, cb2, cln2_g, cln2_b, cW3, cb3):
    raise NotImplementedError("write your pallas kernel here")


# speedup vs baseline: 1.6989x; 1.6989x over previous
import jax
import jax.numpy as jnp
from jax.experimental import pallas as pl
from jax.experimental.pallas import tpu as pltpu

NEG = -1e30


def _ln(x, g, b, eps=1e-5):
    mu = jnp.mean(x, axis=-1, keepdims=True)
    var = jnp.mean((x - mu) ** 2, axis=-1, keepdims=True)
    return (x - mu) / jnp.sqrt(var + eps) * g + b


def _gelu(x):
    return 0.5 * x * (1.0 + jax.lax.erf(x * 0.7071067811865476))


def _mm_body(x_ref, wrel_ref, wroot_ref, b_ref, h3_ref, root_ref):
    xb = x_ref[...]
    h3_ref[0] = jnp.dot(xb, wrel_ref[0], preferred_element_type=jnp.float32)

    @pl.when(pl.program_id(1) == 0)
    def _():
        root_ref[...] = (
            jnp.dot(xb, wroot_ref[...], preferred_element_type=jnp.float32)
            + b_ref[...]
        )


def _rgcn_mm(x, W_rel, W_root, b, bn):
    n, d = x.shape
    r, _, h = W_rel.shape
    return pl.pallas_call(
        _mm_body,
        grid=(n // bn, r),
        in_specs=[
            pl.BlockSpec((bn, d), lambda i, j: (i, 0)),
            pl.BlockSpec((1, d, h), lambda i, j: (j, 0, 0)),
            pl.BlockSpec((d, h), lambda i, j: (0, 0)),
            pl.BlockSpec((1, h), lambda i, j: (0, 0)),
        ],
        out_specs=[
            pl.BlockSpec((1, bn, h), lambda i, j: (j, i, 0)),
            pl.BlockSpec((bn, h), lambda i, j: (i, 0)),
        ],
        out_shape=[
            jax.ShapeDtypeStruct((r, n, h), jnp.float32),
            jax.ShapeDtypeStruct((n, h), jnp.float32),
        ],
    )(x, W_rel, W_root, b.reshape(1, -1))


def _scatter(tbl, aidx, didx, nrows, init_val, is_max, eblk):
    neb = aidx.shape[0]
    hh = tbl.shape[1]

    def body(aidx_ref, didx_ref, tbl_ref, acc_ref):
        @pl.when(pl.program_id(0) == 0)
        def _():
            acc_ref[...] = jnp.full((nrows, hh), init_val, jnp.float32)

        def step(e, carry):
            s = aidx_ref[0, 0, e]
            d = didx_ref[0, 0, e]
            row = tbl_ref[pl.ds(s, 1), :]
            cur = acc_ref[pl.ds(d, 1), :]
            if is_max:
                acc_ref[pl.ds(d, 1), :] = jnp.maximum(cur, row)
            else:
                acc_ref[pl.ds(d, 1), :] = cur + row
            return carry

        jax.lax.fori_loop(0, eblk, step, 0)

    return pl.pallas_call(
        body,
        grid=(neb,),
        in_specs=[
            pl.BlockSpec((1, 1, eblk), lambda i: (i, 0, 0), memory_space=pltpu.SMEM),
            pl.BlockSpec((1, 1, eblk), lambda i: (i, 0, 0), memory_space=pltpu.SMEM),
            pl.BlockSpec(tbl.shape, lambda i: (0, 0)),
        ],
        out_specs=pl.BlockSpec((nrows, hh), lambda i: (0, 0)),
        out_shape=jax.ShapeDtypeStruct((nrows, hh), jnp.float32),
        compiler_params=pltpu.CompilerParams(
            vmem_limit_bytes=100 * 1024 * 1024
        ),
    )(aidx, didx, tbl)


def _epi1_body(root_ref, acc_ref, g_ref, b_ref, out_ref):
    o = root_ref[...]
    racc = acc_ref[...]
    for rr in range(racc.shape[0]):
        a = racc[rr]
        o = o + jnp.where(a <= NEG * 0.5, 0.0, a)
    out_ref[...] = _gelu(_ln(o, g_ref[...], b_ref[...]))


def _epi1(root, acc3, g, b, bn):
    r, n, h = acc3.shape
    return pl.pallas_call(
        _epi1_body,
        grid=(n // bn,),
        in_specs=[
            pl.BlockSpec((bn, h), lambda i: (i, 0)),
            pl.BlockSpec((r, bn, h), lambda i: (0, i, 0)),
            pl.BlockSpec((1, h), lambda i: (0, 0)),
            pl.BlockSpec((1, h), lambda i: (0, 0)),
        ],
        out_specs=pl.BlockSpec((bn, h), lambda i: (i, 0)),
        out_shape=jax.ShapeDtypeStruct((n, h), jnp.float32),
    )(root, acc3, g.reshape(1, -1), b.reshape(1, -1))


def _final_body(
    root2_ref, acc2_ref, x1_ref, g2_ref, bb2_ref,
    cw1_ref, cb1_ref, g3_ref, b3_ref,
    cw2_ref, cb2_ref, g4_ref, b4_ref,
    cw3_ref, cb3_ref, out_ref,
):
    h = _gelu(_ln(root2_ref[...] + acc2_ref[...], g2_ref[...], bb2_ref[...]))
    h = h + x1_ref[...] * 0.5
    h = _gelu(_ln(
        jnp.dot(h, cw1_ref[...], preferred_element_type=jnp.float32) + cb1_ref[...],
        g3_ref[...], b3_ref[...]))
    h = _gelu(_ln(
        jnp.dot(h, cw2_ref[...], preferred_element_type=jnp.float32) + cb2_ref[...],
        g4_ref[...], b4_ref[...]))
    o = jnp.dot(h, cw3_ref[...], preferred_element_type=jnp.float32) + cb3_ref[...]
    m = jnp.max(o, axis=-1, keepdims=True)
    lse = jnp.log(jnp.sum(jnp.exp(o - m), axis=-1, keepdims=True)) + m
    out_ref[...] = o - lse


def _final(root2, acc2, x1, g2, b2v, cW1, cb1, g3, b3, cW2, cb2, g4, b4,
           cW3, cb3, bn):
    n, h = root2.shape
    hm = cW2.shape[1]
    out_dim = cW3.shape[1]

    def vspec(v):
        return pl.BlockSpec((1, v.shape[0]), lambda i: (0, 0))

    def wspec(w):
        return pl.BlockSpec(w.shape, lambda i: (0, 0))

    nblk = pl.BlockSpec((bn, h), lambda i: (i, 0))
    return pl.pallas_call(
        _final_body,
        grid=(n // bn,),
        in_specs=[
            nblk, nblk, nblk,
            vspec(g2), vspec(b2v),
            wspec(cW1), vspec(cb1), vspec(g3), vspec(b3),
            wspec(cW2), vspec(cb2), vspec(g4), vspec(b4),
            wspec(cW3), vspec(cb3),
        ],
        out_specs=pl.BlockSpec((bn, out_dim), lambda i: (i, 0)),
        out_shape=jax.ShapeDtypeStruct((n, out_dim), jnp.float32),
    )(
        root2, acc2, x1,
        g2.reshape(1, -1), b2v.reshape(1, -1),
        cW1, cb1.reshape(1, -1), g3.reshape(1, -1), b3.reshape(1, -1),
        cW2, cb2.reshape(1, -1), g4.reshape(1, -1), b4.reshape(1, -1),
        cW3, cb3.reshape(1, -1),
    )


def kernel(x, edge_index, edge_type, W_rel1, W_root1, b1, ln1_g, ln1_b,
           W_rel2, W_root2, b2, ln2_g, ln2_b, cW1, cb1, cln1_g, cln1_b,
           cW2, cb2, cln2_g, cln2_b, cW3, cb3):
    n, _ = x.shape
    r = W_rel1.shape[0]
    h = W_root1.shape[1]
    e = edge_type.shape[0]
    bn = 1000 if n % 1000 == 0 else n
    eblk = 2000 if e % 2000 == 0 else e

    src = edge_index[0]
    dst = edge_index[1]
    et = edge_type
    gidx = (et * n + src).reshape(-1, 1, eblk)
    didx_max = (et * n + dst).reshape(-1, 1, eblk)
    didx_add = dst.reshape(-1, 1, eblk)

    h3, root1 = _rgcn_mm(x, W_rel1, W_root1, b1, bn)
    accm = _scatter(h3.reshape(r * n, h), gidx, didx_max, r * n, NEG, True, eblk)
    x1 = _epi1(root1, accm.reshape(r, n, h), ln1_g, ln1_b, bn)

    h3b, root2 = _rgcn_mm(x1, W_rel2, W_root2, b2, bn)
    acca = _scatter(h3b.reshape(r * n, h), gidx, didx_add, n, 0.0, False, eblk)

    return _final(root2, acca, x1, ln2_g, ln2_b, cW1, cb1, cln1_g, cln1_b,
                  cW2, cb2, cln2_g, cln2_b, cW3, cb3, bn)


# speedup vs baseline: 2.9005x; 1.7073x over previous
import functools

import jax
import jax.numpy as jnp
from jax.experimental import pallas as pl
from jax.experimental.pallas import tpu as pltpu
from jax.experimental.pallas import tpu_sc as plsc

NEG = -1e30
_SC_CORES = 2
_SC_SUBCORES = 16
_SC_CHUNK = 80


def _sc_scatter_add(tbl, gidx, didx, zinit):
    e = gidx.shape[0]
    n, h = zinit.shape
    nw = _SC_CORES * _SC_SUBCORES
    ew = e // nw
    nchunk = ew // _SC_CHUNK
    mesh = plsc.VectorSubcoreMesh(core_axis_name="c", subcore_axis_name="s")

    @functools.partial(
        pl.kernel,
        out_type=jax.ShapeDtypeStruct((_SC_CORES, n, h), jnp.float32),
        mesh=mesh,
        scratch_types=[
            pltpu.VMEM((_SC_CHUNK,), jnp.int32),
            pltpu.VMEM((_SC_CHUNK,), jnp.int32),
            pltpu.VMEM((_SC_CHUNK, h), jnp.float32),
            pltpu.VMEM_SHARED((n, h), jnp.float32),
            pltpu.SemaphoreType.DMA,
        ],
    )
    def body(tbl_hbm, gidx_hbm, didx_hbm, z_hbm, out_hbm,
             gi_v, di_v, rows_v, acc_sh, sem):
        c = jax.lax.axis_index("c")
        s = jax.lax.axis_index("s")
        wid = s * _SC_CORES + c

        @pl.when(s == 0)
        def _():
            pltpu.sync_copy(z_hbm, acc_sh)

        plsc.subcore_barrier()
        base = wid * ew

        @pl.loop(0, nchunk)
        def _(i):
            off = base + i * _SC_CHUNK
            pltpu.sync_copy(gidx_hbm.at[pl.ds(off, _SC_CHUNK)], gi_v)
            pltpu.sync_copy(didx_hbm.at[pl.ds(off, _SC_CHUNK)], di_v)
            pltpu.async_copy(tbl_hbm.at[gi_v], rows_v, sem).wait()
            pltpu.sync_copy(rows_v, acc_sh.at[di_v], add=True)

        plsc.subcore_barrier()

        @pl.when(s == 0)
        def _():
            pltpu.sync_copy(acc_sh, out_hbm.at[c])

    return body(tbl, gidx, didx, zinit)


def _ln(x, g, b, eps=1e-5):
    mu = jnp.mean(x, axis=-1, keepdims=True)
    var = jnp.mean((x - mu) ** 2, axis=-1, keepdims=True)
    return (x - mu) / jnp.sqrt(var + eps) * g + b


def _gelu(x):
    return 0.5 * x * (1.0 + jax.lax.erf(x * 0.7071067811865476))


def _mm_body(x_ref, wrel_ref, wroot_ref, b_ref, h3_ref, root_ref):
    xb = x_ref[...]
    h3_ref[0] = jnp.dot(xb, wrel_ref[0], preferred_element_type=jnp.float32)

    @pl.when(pl.program_id(1) == 0)
    def _():
        root_ref[...] = (
            jnp.dot(xb, wroot_ref[...], preferred_element_type=jnp.float32)
            + b_ref[...]
        )


def _rgcn_mm(x, W_rel, W_root, b, bn):
    n, d = x.shape
    r, _, h = W_rel.shape
    return pl.pallas_call(
        _mm_body,
        grid=(n // bn, r),
        in_specs=[
            pl.BlockSpec((bn, d), lambda i, j: (i, 0)),
            pl.BlockSpec((1, d, h), lambda i, j: (j, 0, 0)),
            pl.BlockSpec((d, h), lambda i, j: (0, 0)),
            pl.BlockSpec((1, h), lambda i, j: (0, 0)),
        ],
        out_specs=[
            pl.BlockSpec((1, bn, h), lambda i, j: (j, i, 0)),
            pl.BlockSpec((bn, h), lambda i, j: (i, 0)),
        ],
        out_shape=[
            jax.ShapeDtypeStruct((r, n, h), jnp.float32),
            jax.ShapeDtypeStruct((n, h), jnp.float32),
        ],
    )(x, W_rel, W_root, b.reshape(1, -1))


def _scatter(tbl, aidx, didx, nrows, init_val, is_max, eblk):
    neb = aidx.shape[0]
    hh = tbl.shape[1]

    def body(aidx_ref, didx_ref, tbl_ref, acc_ref):
        @pl.when(pl.program_id(0) == 0)
        def _():
            acc_ref[...] = jnp.full((nrows, hh), init_val, jnp.float32)

        def step(e, carry):
            s = aidx_ref[0, 0, e]
            d = didx_ref[0, 0, e]
            row = tbl_ref[pl.ds(s, 1), :]
            cur = acc_ref[pl.ds(d, 1), :]
            if is_max:
                acc_ref[pl.ds(d, 1), :] = jnp.maximum(cur, row)
            else:
                acc_ref[pl.ds(d, 1), :] = cur + row
            return carry

        jax.lax.fori_loop(0, eblk, step, 0)

    return pl.pallas_call(
        body,
        grid=(neb,),
        in_specs=[
            pl.BlockSpec((1, 1, eblk), lambda i: (i, 0, 0), memory_space=pltpu.SMEM),
            pl.BlockSpec((1, 1, eblk), lambda i: (i, 0, 0), memory_space=pltpu.SMEM),
            pl.BlockSpec(tbl.shape, lambda i: (0, 0)),
        ],
        out_specs=pl.BlockSpec((nrows, hh), lambda i: (0, 0)),
        out_shape=jax.ShapeDtypeStruct((nrows, hh), jnp.float32),
        compiler_params=pltpu.CompilerParams(
            vmem_limit_bytes=100 * 1024 * 1024
        ),
    )(aidx, didx, tbl)


def _epi1_body(root_ref, acc_ref, g_ref, b_ref, out_ref):
    o = root_ref[...]
    racc = acc_ref[...]
    for rr in range(racc.shape[0]):
        a = racc[rr]
        o = o + jnp.where(a <= NEG * 0.5, 0.0, a)
    out_ref[...] = _gelu(_ln(o, g_ref[...], b_ref[...]))


def _epi1(root, acc3, g, b, bn):
    r, n, h = acc3.shape
    return pl.pallas_call(
        _epi1_body,
        grid=(n // bn,),
        in_specs=[
            pl.BlockSpec((bn, h), lambda i: (i, 0)),
            pl.BlockSpec((r, bn, h), lambda i: (0, i, 0)),
            pl.BlockSpec((1, h), lambda i: (0, 0)),
            pl.BlockSpec((1, h), lambda i: (0, 0)),
        ],
        out_specs=pl.BlockSpec((bn, h), lambda i: (i, 0)),
        out_shape=jax.ShapeDtypeStruct((n, h), jnp.float32),
    )(root, acc3, g.reshape(1, -1), b.reshape(1, -1))


def _final_body(
    root2_ref, acc2_ref, x1_ref, g2_ref, bb2_ref,
    cw1_ref, cb1_ref, g3_ref, b3_ref,
    cw2_ref, cb2_ref, g4_ref, b4_ref,
    cw3_ref, cb3_ref, out_ref,
):
    agg = acc2_ref[...]
    o = root2_ref[...]
    for kk in range(agg.shape[0]):
        o = o + agg[kk]
    h = _gelu(_ln(o, g2_ref[...], bb2_ref[...]))
    h = h + x1_ref[...] * 0.5
    h = _gelu(_ln(
        jnp.dot(h, cw1_ref[...], preferred_element_type=jnp.float32) + cb1_ref[...],
        g3_ref[...], b3_ref[...]))
    h = _gelu(_ln(
        jnp.dot(h, cw2_ref[...], preferred_element_type=jnp.float32) + cb2_ref[...],
        g4_ref[...], b4_ref[...]))
    o = jnp.dot(h, cw3_ref[...], preferred_element_type=jnp.float32) + cb3_ref[...]
    m = jnp.max(o, axis=-1, keepdims=True)
    lse = jnp.log(jnp.sum(jnp.exp(o - m), axis=-1, keepdims=True)) + m
    out_ref[...] = o - lse


def _final(root2, acc2, x1, g2, b2v, cW1, cb1, g3, b3, cW2, cb2, g4, b4,
           cW3, cb3, bn):
    n, h = root2.shape
    k = acc2.shape[0]
    out_dim = cW3.shape[1]

    def vspec(v):
        return pl.BlockSpec((1, v.shape[0]), lambda i: (0, 0))

    def wspec(w):
        return pl.BlockSpec(w.shape, lambda i: (0, 0))

    nblk = pl.BlockSpec((bn, h), lambda i: (i, 0))
    return pl.pallas_call(
        _final_body,
        grid=(n // bn,),
        in_specs=[
            nblk,
            pl.BlockSpec((k, bn, h), lambda i: (0, i, 0)),
            nblk,
            vspec(g2), vspec(b2v),
            wspec(cW1), vspec(cb1), vspec(g3), vspec(b3),
            wspec(cW2), vspec(cb2), vspec(g4), vspec(b4),
            wspec(cW3), vspec(cb3),
        ],
        out_specs=pl.BlockSpec((bn, out_dim), lambda i: (i, 0)),
        out_shape=jax.ShapeDtypeStruct((n, out_dim), jnp.float32),
    )(
        root2, acc2, x1,
        g2.reshape(1, -1), b2v.reshape(1, -1),
        cW1, cb1.reshape(1, -1), g3.reshape(1, -1), b3.reshape(1, -1),
        cW2, cb2.reshape(1, -1), g4.reshape(1, -1), b4.reshape(1, -1),
        cW3, cb3.reshape(1, -1),
    )


def kernel(x, edge_index, edge_type, W_rel1, W_root1, b1, ln1_g, ln1_b,
           W_rel2, W_root2, b2, ln2_g, ln2_b, cW1, cb1, cln1_g, cln1_b,
           cW2, cb2, cln2_g, cln2_b, cW3, cb3):
    n, _ = x.shape
    r = W_rel1.shape[0]
    h = W_root1.shape[1]
    e = edge_type.shape[0]
    bn = 1000 if n % 1000 == 0 else n
    eblk = 2000 if e % 2000 == 0 else e

    src = edge_index[0]
    dst = edge_index[1]
    et = edge_type
    gidx_f = et * n + src
    gidx = gidx_f.reshape(-1, 1, eblk)
    didx_max = (et * n + dst).reshape(-1, 1, eblk)

    h3, root1 = _rgcn_mm(x, W_rel1, W_root1, b1, bn)
    accm = _scatter(h3.reshape(r * n, h), gidx, didx_max, r * n, NEG, True, eblk)
    x1 = _epi1(root1, accm.reshape(r, n, h), ln1_g, ln1_b, bn)

    h3b, root2 = _rgcn_mm(x1, W_rel2, W_root2, b2, bn)
    if e % (_SC_CORES * _SC_SUBCORES * _SC_CHUNK) == 0:
        acca = _sc_scatter_add(h3b.reshape(r * n, h), gidx_f, dst,
                               jnp.zeros((n, h), jnp.float32))
    else:
        acca = _scatter(h3b.reshape(r * n, h), gidx, dst.reshape(-1, 1, eblk),
                        n, 0.0, False, eblk)[None]

    return _final(root2, acca, x1, ln2_g, ln2_b, cW1, cb1, cln1_g, cln1_b,
                  cW2, cb2, cln2_g, cln2_b, cW3, cb3, bn)


# speedup vs baseline: 4.6314x; 1.5968x over previous
import functools

import jax
import jax.numpy as jnp
from jax.experimental import pallas as pl
from jax.experimental.pallas import tpu as pltpu
from jax.experimental.pallas import tpu_sc as plsc

NEG = -1e30
_SC_CORES = 2
_SC_SUBCORES = 16
_SC_CHUNK = 80


def _sc_scatter_add(tbl, gidx, didx, zinit):
    e = gidx.shape[0]
    n, h = zinit.shape
    nw = _SC_CORES * _SC_SUBCORES
    ew = e // nw
    nchunk = ew // _SC_CHUNK
    mesh = plsc.VectorSubcoreMesh(core_axis_name="c", subcore_axis_name="s")

    @functools.partial(
        pl.kernel,
        out_type=jax.ShapeDtypeStruct((_SC_CORES, n, h), jnp.float32),
        mesh=mesh,
        scratch_types=[
            pltpu.VMEM((_SC_CHUNK,), jnp.int32),
            pltpu.VMEM((_SC_CHUNK,), jnp.int32),
            pltpu.VMEM((_SC_CHUNK, h), jnp.float32),
            pltpu.VMEM_SHARED((n, h), jnp.float32),
            pltpu.SemaphoreType.DMA,
        ],
    )
    def body(tbl_hbm, gidx_hbm, didx_hbm, z_hbm, out_hbm,
             gi_v, di_v, rows_v, acc_sh, sem):
        c = jax.lax.axis_index("c")
        s = jax.lax.axis_index("s")
        wid = s * _SC_CORES + c

        @pl.when(s == 0)
        def _():
            pltpu.sync_copy(z_hbm, acc_sh)

        plsc.subcore_barrier()
        base = wid * ew

        @pl.loop(0, nchunk)
        def _(i):
            off = base + i * _SC_CHUNK
            pltpu.sync_copy(gidx_hbm.at[pl.ds(off, _SC_CHUNK)], gi_v)
            pltpu.sync_copy(didx_hbm.at[pl.ds(off, _SC_CHUNK)], di_v)
            pltpu.async_copy(tbl_hbm.at[gi_v], rows_v, sem).wait()
            pltpu.sync_copy(rows_v, acc_sh.at[di_v], add=True)

        plsc.subcore_barrier()

        @pl.when(s == 0)
        def _():
            pltpu.sync_copy(acc_sh, out_hbm.at[c])

    return body(tbl, gidx, didx, zinit)


def _ln(x, g, b, eps=1e-5):
    mu = jnp.mean(x, axis=-1, keepdims=True)
    var = jnp.mean((x - mu) ** 2, axis=-1, keepdims=True)
    return (x - mu) / jnp.sqrt(var + eps) * g + b


def _gelu(x):
    return 0.5 * x * (1.0 + jax.lax.erf(x * 0.7071067811865476))


def _mm_body(x_ref, wrel_ref, wroot_ref, b_ref, h3_ref, root_ref):
    xb = x_ref[...]
    h3_ref[0] = jnp.dot(xb, wrel_ref[0], preferred_element_type=jnp.float32)

    @pl.when(pl.program_id(1) == 0)
    def _():
        root_ref[...] = (
            jnp.dot(xb, wroot_ref[...], preferred_element_type=jnp.float32)
            + b_ref[...]
        )


def _rgcn_mm(x, W_rel, W_root, b, bn):
    n, d = x.shape
    r, _, h = W_rel.shape
    return pl.pallas_call(
        _mm_body,
        grid=(n // bn, r),
        in_specs=[
            pl.BlockSpec((bn, d), lambda i, j: (i, 0)),
            pl.BlockSpec((1, d, h), lambda i, j: (j, 0, 0)),
            pl.BlockSpec((d, h), lambda i, j: (0, 0)),
            pl.BlockSpec((1, h), lambda i, j: (0, 0)),
        ],
        out_specs=[
            pl.BlockSpec((1, bn, h), lambda i, j: (j, i, 0)),
            pl.BlockSpec((bn, h), lambda i, j: (i, 0)),
        ],
        out_shape=[
            jax.ShapeDtypeStruct((r, n, h), jnp.float32),
            jax.ShapeDtypeStruct((n, h), jnp.float32),
        ],
    )(x, W_rel, W_root, b.reshape(1, -1))


def _scatter(tbl, aidx, didx, nrows, init_val, is_max, eblk):
    neb = aidx.shape[0]
    hh = tbl.shape[1]

    def body(aidx_ref, didx_ref, tbl_ref, acc_ref):
        @pl.when(pl.program_id(0) == 0)
        def _():
            acc_ref[...] = jnp.full((nrows, hh), init_val, jnp.float32)

        def step(e, carry):
            s = aidx_ref[0, 0, e]
            d = didx_ref[0, 0, e]
            row = tbl_ref[pl.ds(s, 1), :]
            cur = acc_ref[pl.ds(d, 1), :]
            if is_max:
                acc_ref[pl.ds(d, 1), :] = jnp.maximum(cur, row)
            else:
                acc_ref[pl.ds(d, 1), :] = cur + row
            return carry

        jax.lax.fori_loop(0, eblk, step, 0, unroll=8)

    return pl.pallas_call(
        body,
        grid=(neb,),
        in_specs=[
            pl.BlockSpec((1, 1, eblk), lambda i: (i, 0, 0), memory_space=pltpu.SMEM),
            pl.BlockSpec((1, 1, eblk), lambda i: (i, 0, 0), memory_space=pltpu.SMEM),
            pl.BlockSpec(tbl.shape, lambda i: (0, 0)),
        ],
        out_specs=pl.BlockSpec((nrows, hh), lambda i: (0, 0)),
        out_shape=jax.ShapeDtypeStruct((nrows, hh), jnp.float32),
        compiler_params=pltpu.CompilerParams(
            vmem_limit_bytes=100 * 1024 * 1024
        ),
    )(aidx, didx, tbl)


def _epi1_body(root_ref, acc_ref, g_ref, b_ref, out_ref):
    o = root_ref[...]
    racc = acc_ref[...]
    for rr in range(racc.shape[0]):
        a = racc[rr]
        o = o + jnp.where(a <= NEG * 0.5, 0.0, a)
    out_ref[...] = _gelu(_ln(o, g_ref[...], b_ref[...]))


def _epi1(root, acc3, g, b, bn):
    r, n, h = acc3.shape
    return pl.pallas_call(
        _epi1_body,
        grid=(n // bn,),
        in_specs=[
            pl.BlockSpec((bn, h), lambda i: (i, 0)),
            pl.BlockSpec((r, bn, h), lambda i: (0, i, 0)),
            pl.BlockSpec((1, h), lambda i: (0, 0)),
            pl.BlockSpec((1, h), lambda i: (0, 0)),
        ],
        out_specs=pl.BlockSpec((bn, h), lambda i: (i, 0)),
        out_shape=jax.ShapeDtypeStruct((n, h), jnp.float32),
    )(root, acc3, g.reshape(1, -1), b.reshape(1, -1))


def _final_body(
    root2_ref, acc2_ref, x1_ref, g2_ref, bb2_ref,
    cw1_ref, cb1_ref, g3_ref, b3_ref,
    cw2_ref, cb2_ref, g4_ref, b4_ref,
    cw3_ref, cb3_ref, out_ref,
):
    agg = acc2_ref[...]
    o = root2_ref[...]
    for kk in range(agg.shape[0]):
        o = o + agg[kk]
    h = _gelu(_ln(o, g2_ref[...], bb2_ref[...]))
    h = h + x1_ref[...] * 0.5
    h = _gelu(_ln(
        jnp.dot(h, cw1_ref[...], preferred_element_type=jnp.float32) + cb1_ref[...],
        g3_ref[...], b3_ref[...]))
    h = _gelu(_ln(
        jnp.dot(h, cw2_ref[...], preferred_element_type=jnp.float32) + cb2_ref[...],
        g4_ref[...], b4_ref[...]))
    o = jnp.dot(h, cw3_ref[...], preferred_element_type=jnp.float32) + cb3_ref[...]
    m = jnp.max(o, axis=-1, keepdims=True)
    lse = jnp.log(jnp.sum(jnp.exp(o - m), axis=-1, keepdims=True)) + m
    out_ref[...] = o - lse


def _final(root2, acc2, x1, g2, b2v, cW1, cb1, g3, b3, cW2, cb2, g4, b4,
           cW3, cb3, bn):
    n, h = root2.shape
    k = acc2.shape[0]
    out_dim = cW3.shape[1]

    def vspec(v):
        return pl.BlockSpec((1, v.shape[0]), lambda i: (0, 0))

    def wspec(w):
        return pl.BlockSpec(w.shape, lambda i: (0, 0))

    nblk = pl.BlockSpec((bn, h), lambda i: (i, 0))
    return pl.pallas_call(
        _final_body,
        grid=(n // bn,),
        in_specs=[
            nblk,
            pl.BlockSpec((k, bn, h), lambda i: (0, i, 0)),
            nblk,
            vspec(g2), vspec(b2v),
            wspec(cW1), vspec(cb1), vspec(g3), vspec(b3),
            wspec(cW2), vspec(cb2), vspec(g4), vspec(b4),
            wspec(cW3), vspec(cb3),
        ],
        out_specs=pl.BlockSpec((bn, out_dim), lambda i: (i, 0)),
        out_shape=jax.ShapeDtypeStruct((n, out_dim), jnp.float32),
    )(
        root2, acc2, x1,
        g2.reshape(1, -1), b2v.reshape(1, -1),
        cW1, cb1.reshape(1, -1), g3.reshape(1, -1), b3.reshape(1, -1),
        cW2, cb2.reshape(1, -1), g4.reshape(1, -1), b4.reshape(1, -1),
        cW3, cb3.reshape(1, -1),
    )


def kernel(x, edge_index, edge_type, W_rel1, W_root1, b1, ln1_g, ln1_b,
           W_rel2, W_root2, b2, ln2_g, ln2_b, cW1, cb1, cln1_g, cln1_b,
           cW2, cb2, cln2_g, cln2_b, cW3, cb3):
    n, _ = x.shape
    r = W_rel1.shape[0]
    h = W_root1.shape[1]
    e = edge_type.shape[0]
    bn = 1000 if n % 1000 == 0 else n
    eblk = 2000 if e % 2000 == 0 else e

    src = edge_index[0]
    dst = edge_index[1]
    et = edge_type
    gidx_f = et * n + src
    gidx = gidx_f.reshape(-1, 1, eblk)
    didx_max = (et * n + dst).reshape(-1, 1, eblk)

    h3, root1 = _rgcn_mm(x, W_rel1, W_root1, b1, bn)
    accm = _scatter(h3.reshape(r * n, h), gidx, didx_max, r * n, NEG, True, eblk)
    x1 = _epi1(root1, accm.reshape(r, n, h), ln1_g, ln1_b, bn)

    h3b, root2 = _rgcn_mm(x1, W_rel2, W_root2, b2, bn)
    if e % (_SC_CORES * _SC_SUBCORES * _SC_CHUNK) == 0:
        acca = _sc_scatter_add(h3b.reshape(r * n, h), gidx_f, dst,
                               jnp.zeros((n, h), jnp.float32))
    else:
        acca = _scatter(h3b.reshape(r * n, h), gidx, dst.reshape(-1, 1, eblk),
                        n, 0.0, False, eblk)[None]

    return _final(root2, acca, x1, ln2_g, ln2_b, cW1, cb1, cln1_g, cln1_b,
                  cW2, cb2, cln2_g, cln2_b, cW3, cb3, bn)
